# baseline (speedup 1.0000x reference)
import jax
import jax.numpy as jnp
from jax.experimental import pallas as pl
from jax.experimental.pallas import tpu as pltpu


def _fused_body(x_ref, w_ref, o_ref):
    B, P, D = x_ref.shape
    C = w_ref.shape[0]
    xr = x_ref[...].reshape(B * P, D)
    w = w_ref[...]
    hT = jax.lax.dot_general(
        w, xr, (((1,), (1,)), ((), ())), preferred_element_type=jnp.float32
    )
    iota_l = jax.lax.broadcasted_iota(jnp.int32, (C, P), 1)
    segs = []
    for b in range(B):
        seg = jax.lax.slice(hT, (0, b * P), (C, (b + 1) * P))
        m = jnp.max(seg, axis=1, keepdims=True)
        first = jnp.min(jnp.where(seg == m, iota_l, P), axis=1, keepdims=True)
        idx = jnp.where(m > 0, first, 0)
        segs.append((iota_l == idx).astype(jnp.bfloat16))
    onehotT = jax.lax.concatenate(segs, 1)
    out = jax.lax.dot_general(
        onehotT, w.astype(jnp.bfloat16), (((0,), (0,)), ((), ())),
        preferred_element_type=jnp.float32,
    )
    o_ref[...] = out.reshape(B, P, D)


def kernel(x, W):
    bsz, P, d = x.shape
    C = W.shape[0]
    B_BLK = 16
    grid = (bsz // B_BLK,)
    return pl.pallas_call(
        _fused_body,
        grid=grid,
        in_specs=[
            pl.BlockSpec((B_BLK, P, d), lambda i: (i, 0, 0)),
            pl.BlockSpec((C, d), lambda i: (0, 0)),
        ],
        out_specs=pl.BlockSpec((B_BLK, P, d), lambda i: (i, 0, 0)),
        out_shape=jax.ShapeDtypeStruct((bsz, P, d), x.dtype),
        compiler_params=pltpu.CompilerParams(
            dimension_semantics=("parallel",),
        ),
    )(x, W)


# speedup vs baseline: 1.0936x; 1.0936x over previous
import jax
import jax.numpy as jnp
from jax.experimental import pallas as pl
from jax.experimental.pallas import tpu as pltpu


def _fused_body(x_ref, w_ref, o_ref):
    B, P, D = x_ref.shape
    C = w_ref.shape[0]
    xr = x_ref[...].reshape(B * P, D)
    w = w_ref[...]
    hT = jax.lax.dot_general(
        w, xr, (((1,), (1,)), ((), ())), preferred_element_type=jnp.float32
    )
    iota_l = jax.lax.broadcasted_iota(jnp.int32, (C, P), 1)
    segs = []
    for b in range(B):
        seg = jax.lax.slice(hT, (0, b * P), (C, (b + 1) * P))
        m = jnp.max(seg, axis=1, keepdims=True)
        first = jnp.min(jnp.where(seg == m, iota_l, P), axis=1, keepdims=True)
        idx = jnp.where(m > 0, first, 0)
        segs.append((iota_l == idx).astype(jnp.bfloat16))
    onehotT = jax.lax.concatenate(segs, 1)
    out = jax.lax.dot_general(
        onehotT, w.astype(jnp.bfloat16), (((0,), (0,)), ((), ())),
        preferred_element_type=jnp.float32,
    )
    o_ref[...] = out.reshape(B, P, D)


def kernel(x, W):
    bsz, P, d = x.shape
    C = W.shape[0]
    B_BLK = 32
    grid = (bsz // B_BLK,)
    return pl.pallas_call(
        _fused_body,
        grid=grid,
        in_specs=[
            pl.BlockSpec((B_BLK, P, d), lambda i: (i, 0, 0)),
            pl.BlockSpec((C, d), lambda i: (0, 0)),
        ],
        out_specs=pl.BlockSpec((B_BLK, P, d), lambda i: (i, 0, 0)),
        out_shape=jax.ShapeDtypeStruct((bsz, P, d), x.dtype),
        compiler_params=pltpu.CompilerParams(
            dimension_semantics=("parallel",),
        ),
    )(x, W)
